# baseline (speedup 1.0000x reference)
import jax
import jax.numpy as jnp
from jax.experimental import pallas as pl
from jax.experimental.pallas import tpu as pltpu

N_CAPS = 4
D_FEAT = 8
GROUP = 4


def _make_body(nb, R):
    fold_rows = (R + 3) // 4
    stream_rows = (R * GROUP * D_FEAT) // 128

    def body(p_ref, w_ref, b_ref, o_ref):
        row_id = jax.lax.broadcasted_iota(jnp.int32, (fold_rows, GROUP * D_FEAT), 0)
        last = fold_rows - 1
        for i in range(nb):
            p_full = p_ref[i]
            kb = p_full.shape[1] // GROUP
            blocks = [p_full[:, v * kb:(v + 1) * kb] for v in range(GROUP)]
            y = None
            for c in range(N_CAPS):
                parts = []
                for u in range(GROUP):
                    v = (u - c) % GROUP
                    acc = jnp.dot(blocks[v], w_ref[c],
                                  preferred_element_type=jnp.float32)
                    acc = acc + b_ref[c]
                    if v > 0:
                        acc = jnp.where(row_id < last, acc, 0.0)
                    if u < c:
                        acc = jnp.pad(acc, ((1, 0), (0, 0)))[:fold_rows]
                    parts.append(acc)
                band = jnp.concatenate(parts, axis=1)
                top = stream_rows * c
                contrib = jnp.pad(band, ((top, R - fold_rows - top), (0, 0)))
                y = contrib if y is None else y + contrib
            o_ref[i] = y
    return body


@jax.jit
def _forward(x_nchw, weight_oihw, bias):
    N, Cin, H, W = x_nchw.shape
    Cout, wcin, KH, KW = weight_oihw.shape
    H_out = H - KH + 1
    W_out = W - KW + 1
    HW = H_out * W_out
    Kdim = KH * KW * Cin
    R = HW // GROUP
    fold_rows = (R + 3) // 4
    hw_pad = fold_rows * 16

    x_nhwc = jnp.transpose(x_nchw, (0, 2, 3, 1)).astype(jnp.bfloat16)
    taps = []
    for kh in range(KH):
        for kw in range(KW):
            taps.append(x_nhwc[:, kh:kh + H_out, kw:kw + W_out, :])
    patches = jnp.concatenate(taps, axis=-1).reshape(N, HW, Kdim)
    patches = jnp.pad(patches, ((0, 0), (0, hw_pad - HW), (0, 0)))
    p_view = patches.reshape(N, fold_rows, 16 * Kdim)

    w2d = jnp.transpose(weight_oihw, (2, 3, 1, 0)).reshape(Kdim, Cout)
    base = w2d.reshape(Kdim, N_CAPS, D_FEAT).astype(jnp.float32)
    eye = jnp.eye(GROUP, dtype=jnp.float32)
    w_stack = jnp.einsum("jJ,kcd->cjkJd", eye, base)
    w_stack = w_stack.reshape(N_CAPS, GROUP * Kdim, GROUP * D_FEAT)
    w_stack = w_stack.astype(jnp.bfloat16)

    b2 = bias.astype(jnp.float32).reshape(N_CAPS, 1, 1, D_FEAT)
    b_stack = jnp.broadcast_to(b2, (N_CAPS, 1, GROUP, D_FEAT))
    b_stack = b_stack.reshape(N_CAPS, 1, GROUP * D_FEAT)

    nb = 32 if N % 32 == 0 else 1
    grid = (N // nb,)

    out = pl.pallas_call(
        _make_body(nb, R),
        out_shape=jax.ShapeDtypeStruct((N, R, 128), jnp.float32),
        grid=grid,
        in_specs=[
            pl.BlockSpec((nb, fold_rows, 4 * GROUP * Kdim), lambda i: (i, 0, 0)),
            pl.BlockSpec((N_CAPS, GROUP * Kdim, GROUP * D_FEAT), lambda i: (0, 0, 0)),
            pl.BlockSpec((N_CAPS, 1, GROUP * D_FEAT), lambda i: (0, 0, 0)),
        ],
        out_specs=pl.BlockSpec((nb, R, 128), lambda i: (i, 0, 0)),
        compiler_params=pltpu.CompilerParams(dimension_semantics=("parallel",)),
    )(p_view, w_stack, b_stack)

    return out.reshape(N, N_CAPS * HW, D_FEAT).astype(x_nchw.dtype)


def kernel(x_nchw, weight_oihw, bias):
    return _forward(x_nchw, weight_oihw, bias)


# speedup vs baseline: 1.0006x; 1.0006x over previous
import jax
import jax.numpy as jnp
from jax.experimental import pallas as pl
from jax.experimental.pallas import tpu as pltpu

N_CAPS = 4
D_FEAT = 8
GROUP = 4


def _make_body(nb, R):
    fold_rows = (R + 3) // 4
    stream_rows = (R * GROUP * D_FEAT) // 128

    def body(p_ref, w_ref, b_ref, o_ref):
        row_id = jax.lax.broadcasted_iota(jnp.int32, (fold_rows, GROUP * D_FEAT), 0)
        last = fold_rows - 1
        for i in range(nb):
            p_full = p_ref[i]
            kb = p_full.shape[1] // GROUP
            blocks = [p_full[:, v * kb:(v + 1) * kb] for v in range(GROUP)]
            y = None
            for c in range(N_CAPS):
                parts = []
                for u in range(GROUP):
                    v = (u - c) % GROUP
                    acc = jnp.dot(blocks[v], w_ref[c],
                                  preferred_element_type=jnp.float32)
                    acc = acc + b_ref[c]
                    if v > 0:
                        acc = jnp.where(row_id < last, acc, 0.0)
                    if u < c:
                        acc = jnp.pad(acc, ((1, 0), (0, 0)))[:fold_rows]
                    parts.append(acc)
                band = jnp.concatenate(parts, axis=1)
                top = stream_rows * c
                contrib = jnp.pad(band, ((top, R - fold_rows - top), (0, 0)))
                y = contrib if y is None else y + contrib
            o_ref[i] = y
    return body


@jax.jit
def _forward(x_nchw, weight_oihw, bias):
    N, Cin, H, W = x_nchw.shape
    Cout, wcin, KH, KW = weight_oihw.shape
    H_out = H - KH + 1
    W_out = W - KW + 1
    HW = H_out * W_out
    Kdim = KH * KW * Cin
    R = HW // GROUP
    fold_rows = (R + 3) // 4
    hw_pad = fold_rows * 16

    x_nhwc = jnp.transpose(x_nchw, (0, 2, 3, 1)).astype(jnp.bfloat16)
    taps = []
    for kh in range(KH):
        for kw in range(KW):
            taps.append(x_nhwc[:, kh:kh + H_out, kw:kw + W_out, :])
    patches = jnp.concatenate(taps, axis=-1).reshape(N, HW, Kdim)
    patches = jnp.pad(patches, ((0, 0), (0, hw_pad - HW), (0, 0)))
    p_view = patches.reshape(N, fold_rows, 16 * Kdim)

    w2d = jnp.transpose(weight_oihw, (2, 3, 1, 0)).reshape(Kdim, Cout)
    base = w2d.reshape(Kdim, N_CAPS, D_FEAT).astype(jnp.float32)
    eye = jnp.eye(GROUP, dtype=jnp.float32)
    w_stack = jnp.einsum("jJ,kcd->cjkJd", eye, base)
    w_stack = w_stack.reshape(N_CAPS, GROUP * Kdim, GROUP * D_FEAT)
    w_stack = w_stack.astype(jnp.bfloat16)

    b2 = bias.astype(jnp.float32).reshape(N_CAPS, 1, 1, D_FEAT)
    b_stack = jnp.broadcast_to(b2, (N_CAPS, 1, GROUP, D_FEAT))
    b_stack = b_stack.reshape(N_CAPS, 1, GROUP * D_FEAT)

    nb = 32 if N % 32 == 0 else 1
    grid = (N // nb,)

    out = pl.pallas_call(
        _make_body(nb, R),
        out_shape=jax.ShapeDtypeStruct((N, R, 128), jnp.float32),
        grid=grid,
        in_specs=[
            pl.BlockSpec((nb, fold_rows, 4 * GROUP * Kdim), lambda i: (i, 0, 0)),
            pl.BlockSpec((N_CAPS, GROUP * Kdim, GROUP * D_FEAT), lambda i: (0, 0, 0)),
            pl.BlockSpec((N_CAPS, 1, GROUP * D_FEAT), lambda i: (0, 0, 0)),
        ],
        out_specs=pl.BlockSpec((nb, R, 128), lambda i: (i, 0, 0)),
        compiler_params=pltpu.CompilerParams(dimension_semantics=("arbitrary",)),
    )(p_view, w_stack, b_stack)

    return out.reshape(N, N_CAPS * HW, D_FEAT).astype(x_nchw.dtype)


def kernel(x_nchw, weight_oihw, bias):
    return _forward(x_nchw, weight_oihw, bias)


# speedup vs baseline: 1.0049x; 1.0043x over previous
import jax
import jax.numpy as jnp
from jax.experimental import pallas as pl
from jax.experimental.pallas import tpu as pltpu

N_CAPS = 4
D_FEAT = 8
GROUP = 4


def _make_body(nb, R):
    fold_rows = (R + 3) // 4
    stream_rows = (R * GROUP * D_FEAT) // 128

    def body(p_ref, w_ref, b_ref, o_ref):
        row_id = jax.lax.broadcasted_iota(jnp.int32, (fold_rows, GROUP * D_FEAT), 0)
        last = fold_rows - 1
        for i in range(nb):
            p_full = p_ref[i]
            kb = p_full.shape[1] // GROUP
            blocks = [p_full[:, v * kb:(v + 1) * kb] for v in range(GROUP)]
            y = None
            for c in range(N_CAPS):
                parts = []
                for u in range(GROUP):
                    v = (u - c) % GROUP
                    acc = jnp.dot(blocks[v], w_ref[c],
                                  preferred_element_type=jnp.float32)
                    acc = acc + b_ref[c]
                    if v > 0:
                        acc = jnp.where(row_id < last, acc, 0.0)
                    if u < c:
                        acc = jnp.pad(acc, ((1, 0), (0, 0)))[:fold_rows]
                    parts.append(acc)
                band = jnp.concatenate(parts, axis=1)
                top = stream_rows * c
                contrib = jnp.pad(band, ((top, R - fold_rows - top), (0, 0)))
                y = contrib if y is None else y + contrib
            o_ref[i] = y
    return body


@jax.jit
def _forward(x_nchw, weight_oihw, bias):
    N, Cin, H, W = x_nchw.shape
    Cout, wcin, KH, KW = weight_oihw.shape
    H_out = H - KH + 1
    W_out = W - KW + 1
    HW = H_out * W_out
    Kdim = KH * KW * Cin
    R = HW // GROUP
    fold_rows = (R + 3) // 4
    hw_pad = fold_rows * 16

    x_nhwc = jnp.transpose(x_nchw, (0, 2, 3, 1)).astype(jnp.bfloat16)
    taps = []
    for kh in range(KH):
        for kw in range(KW):
            taps.append(x_nhwc[:, kh:kh + H_out, kw:kw + W_out, :])
    patches = jnp.concatenate(taps, axis=-1).reshape(N, HW, Kdim)
    patches = jnp.pad(patches, ((0, 0), (0, hw_pad - HW), (0, 0)))
    p_view = patches.reshape(N, fold_rows, 16 * Kdim)

    w2d = jnp.transpose(weight_oihw, (2, 3, 1, 0)).reshape(Kdim, Cout)
    base = w2d.reshape(Kdim, N_CAPS, D_FEAT).astype(jnp.float32)
    eye = jnp.eye(GROUP, dtype=jnp.float32)
    w_stack = jnp.einsum("jJ,kcd->cjkJd", eye, base)
    w_stack = w_stack.reshape(N_CAPS, GROUP * Kdim, GROUP * D_FEAT)
    w_stack = w_stack.astype(jnp.bfloat16)

    b2 = bias.astype(jnp.float32).reshape(N_CAPS, 1, 1, D_FEAT)
    b_stack = jnp.broadcast_to(b2, (N_CAPS, 1, GROUP, D_FEAT))
    b_stack = b_stack.reshape(N_CAPS, 1, GROUP * D_FEAT)

    nb = 32 if N % 32 == 0 else 1
    grid = (N // nb,)

    out = pl.pallas_call(
        _make_body(nb, R),
        out_shape=jax.ShapeDtypeStruct((N, R, 128), jnp.float32),
        grid=grid,
        in_specs=[
            pl.BlockSpec((nb, fold_rows, 4 * GROUP * Kdim), lambda i: (i, 0, 0)),
            pl.BlockSpec((N_CAPS, GROUP * Kdim, GROUP * D_FEAT), lambda i: (0, 0, 0)),
            pl.BlockSpec((N_CAPS, 1, GROUP * D_FEAT), lambda i: (0, 0, 0)),
        ],
        out_specs=pl.BlockSpec((nb, R, 128), lambda i: (i, 0, 0)),
        compiler_params=pltpu.CompilerParams(dimension_semantics=("parallel",)),
    )(p_view, w_stack, b_stack)

    return out.reshape(N, N_CAPS * HW, D_FEAT).astype(x_nchw.dtype)


def kernel(x_nchw, weight_oihw, bias):
    return _forward(x_nchw, weight_oihw, bias)
